# baseline (speedup 1.0000x reference)
import functools

import jax
import jax.numpy as jnp
from jax import lax
from jax.experimental import pallas as pl
from jax.experimental.pallas import tpu as pltpu
from jax.experimental.pallas import tpu_sc as plsc

S = 20
A = 24
D = 64
BT = 1024
NC, NS = 2, 16
NW = NC * NS
GR = 128
OP = D * 2 + 1
NBUF = 4
NU = S * A * (BT // GR)


def _make_sc_kernel():
    u_per_w = NU // NW
    mesh = plsc.VectorSubcoreMesh(core_axis_name="c", subcore_axis_name="s")

    @functools.partial(
        pl.kernel,
        out_type=jax.ShapeDtypeStruct((S, A, 8, BT // GR, 8, GR), jnp.float32),
        mesh=mesh,
        compiler_params=pltpu.CompilerParams(use_tc_tiling_on_sc=False,
                                             needs_layout_passes=False),
        scratch_types=[
            pltpu.VMEM((NU // NW, GR), jnp.int32),
            [pltpu.VMEM((GR, D), jnp.float32) for _ in range(NBUF)],
            [pltpu.VMEM((8, 8, GR + 1), jnp.float32) for _ in range(NBUF)],
            pltpu.VMEM((A, D), jnp.float32),
            [pltpu.SemaphoreType.DMA for _ in range(NBUF)],
            [pltpu.SemaphoreType.DMA for _ in range(NBUF)],
        ],
    )
    def body(xl_hbm, emb_hbm, pos_hbm, out_hbm,
             idx_v, g_bufs, o_bufs, pos_v, gsem, osem):
        wid = lax.axis_index("s") * NC + lax.axis_index("c")
        q0 = pl.multiple_of(wid * u_per_w, u_per_w)
        pltpu.sync_copy(pos_hbm.at[pl.ds(0, A)], pos_v)
        pltpu.sync_copy(xl_hbm.at[pl.ds(q0, u_per_w)], idx_v)

        lanes = lax.iota(jnp.int32, 16)
        iv = lanes & 7
        rvecs = [lax.shift_right_logical(lanes, 3) + dg * 2
                 for dg in range(D // 16)]

        def fire_gather(u, k):
            pltpu.async_copy(emb_hbm.at[idx_v.at[u]], g_bufs[k], gsem[k])

        def wait_gather(k):
            pltpu.make_async_copy(emb_hbm.at[pl.ds(0, GR)], g_bufs[k],
                                  gsem[k]).wait()

        def fire_out(u, k):
            q = q0 + u
            s = q // (A * (BT // GR))
            a = (q // (BT // GR)) % A
            c = q % (BT // GR)
            pltpu.async_copy(o_bufs[k].at[:, :, pl.ds(0, GR)],
                             out_hbm.at[s, a, :, c], osem[k])

        def wait_out(k):
            pltpu.make_async_copy(out_hbm.at[0, 0, :, 0],
                                  o_bufs[k].at[:, :, pl.ds(0, GR)],
                                  osem[k]).wait()

        def transpose_add(u, k):
            q = q0 + u
            a = (q // (BT // GR)) % A
            pvecs = [pos_v[a, pl.ds(dg * 16, 16)] for dg in range(D // 16)]

            def b_body(b, _, k=k, pvecs=pvecs):
                cvec = jnp.full((16,), b, jnp.int32)
                for dg in range(D // 16):
                    row = g_bufs[k][b, pl.ds(dg * 16, 16)] + pvecs[dg]
                    plsc.store_scatter(o_bufs[k], [rvecs[dg], iv, cvec], row)
                return 0

            lax.fori_loop(0, GR, b_body, 0, unroll=8)

        for k in range(NBUF - 1):
            fire_gather(k, k)

        def step_body(t, carry):
            for k in range(NBUF):
                u = t * NBUF + k
                @pl.when(u >= NBUF)
                def _(k=k):
                    wait_out(k)

                kn = (k + NBUF - 1) % NBUF

                @pl.when(u + NBUF - 1 < u_per_w)
                def _(u=u, kn=kn):
                    fire_gather(u + NBUF - 1, kn)

                wait_gather(k)
                transpose_add(u, k)
                fire_out(u, k)
            return carry

        lax.fori_loop(0, u_per_w // NBUF, step_body, 0)
        for k in range(NBUF):
            wait_out(k)

    return body


def kernel(x, action_emb, action_pos_emb):
    xl = jnp.transpose(x, (1, 2, 0)).reshape(NU, GR)
    out_t = _make_sc_kernel()(xl, action_emb, action_pos_emb)
    return jnp.transpose(out_t, (3, 5, 0, 1, 2, 4)).reshape(BT, S, A, D)


# speedup vs baseline: 1.0197x; 1.0197x over previous
import functools

import jax
import jax.numpy as jnp
from jax import lax
from jax.experimental import pallas as pl
from jax.experimental.pallas import tpu as pltpu
from jax.experimental.pallas import tpu_sc as plsc

S = 20
A = 24
D = 64
BT = 1024
NC, NS = 2, 16
NW = NC * NS
GR = 128
OP = D * 2 + 1
NBUF = 4
NU = S * A * (BT // GR)


def _make_sc_kernel():
    u_per_w = NU // NW
    mesh = plsc.VectorSubcoreMesh(core_axis_name="c", subcore_axis_name="s")

    @functools.partial(
        pl.kernel,
        out_type=jax.ShapeDtypeStruct((S, A, 8, BT // GR, 8, GR), jnp.float32),
        mesh=mesh,
        compiler_params=pltpu.CompilerParams(use_tc_tiling_on_sc=False,
                                             needs_layout_passes=False),
        scratch_types=[
            pltpu.VMEM((NU // NW, GR), jnp.int32),
            [pltpu.VMEM((GR, D), jnp.float32) for _ in range(NBUF)],
            [pltpu.VMEM((8, 8, GR + 1), jnp.float32) for _ in range(NBUF)],
            pltpu.VMEM((A, D), jnp.float32),
            [pltpu.SemaphoreType.DMA for _ in range(NBUF)],
            [pltpu.SemaphoreType.DMA for _ in range(NBUF)],
        ],
    )
    def body(xl_hbm, emb_hbm, pos_hbm, out_hbm,
             idx_v, g_bufs, o_bufs, pos_v, gsem, osem):
        wid = lax.axis_index("s") * NC + lax.axis_index("c")
        q0 = pl.multiple_of(wid * u_per_w, u_per_w)
        pltpu.sync_copy(pos_hbm.at[pl.ds(0, A)], pos_v)
        pltpu.sync_copy(xl_hbm.at[pl.ds(q0, u_per_w)], idx_v)

        lanes = lax.iota(jnp.int32, 16)
        iv = lanes & 7
        rvecs = [lax.shift_right_logical(lanes, 3) + dg * 2
                 for dg in range(D // 16)]

        def fire_gather(u, k):
            pltpu.async_copy(emb_hbm.at[idx_v.at[u]], g_bufs[k], gsem[k])

        def wait_gather(k):
            pltpu.make_async_copy(emb_hbm.at[pl.ds(0, GR)], g_bufs[k],
                                  gsem[k]).wait()

        def fire_out(u, k):
            q = q0 + u
            s = q // (A * (BT // GR))
            a = (q // (BT // GR)) % A
            c = q % (BT // GR)
            pltpu.async_copy(o_bufs[k].at[:, :, pl.ds(0, GR)],
                             out_hbm.at[s, a, :, c], osem[k])

        def wait_out(k):
            pltpu.make_async_copy(out_hbm.at[0, 0, :, 0],
                                  o_bufs[k].at[:, :, pl.ds(0, GR)],
                                  osem[k]).wait()

        def transpose_add(u, k):
            q = q0 + u
            a = (q // (BT // GR)) % A
            pvecs = [pos_v[a, pl.ds(dg * 16, 16)] for dg in range(D // 16)]

            def b_body(b, _, k=k, pvecs=pvecs):
                cvec = jnp.full((16,), b, jnp.int32)
                for dg in range(D // 16):
                    row = g_bufs[k][b, pl.ds(dg * 16, 16)] + pvecs[dg]
                    plsc.store_scatter(o_bufs[k], [rvecs[dg], iv, cvec], row)
                return 0

            lax.fori_loop(0, GR, b_body, 0, unroll=4)

        for k in range(NBUF - 1):
            fire_gather(k, k)

        def step_body(t, carry):
            for k in range(NBUF):
                u = t * NBUF + k
                @pl.when(u >= NBUF)
                def _(k=k):
                    wait_out(k)

                kn = (k + NBUF - 1) % NBUF

                @pl.when(u + NBUF - 1 < u_per_w)
                def _(u=u, kn=kn):
                    fire_gather(u + NBUF - 1, kn)

                wait_gather(k)
                transpose_add(u, k)
                fire_out(u, k)
            return carry

        lax.fori_loop(0, u_per_w // NBUF, step_body, 0)
        for k in range(NBUF):
            wait_out(k)

    return body


def kernel(x, action_emb, action_pos_emb):
    xl = jnp.transpose(x, (1, 2, 0)).reshape(NU, GR)
    out_t = _make_sc_kernel()(xl, action_emb, action_pos_emb)
    return jnp.transpose(out_t, (3, 5, 0, 1, 2, 4)).reshape(BT, S, A, D)


# speedup vs baseline: 1.0214x; 1.0017x over previous
import functools

import jax
import jax.numpy as jnp
from jax import lax
from jax.experimental import pallas as pl
from jax.experimental.pallas import tpu as pltpu
from jax.experimental.pallas import tpu_sc as plsc

S = 20
A = 24
D = 64
BT = 1024
NC, NS = 2, 16
NW = NC * NS
GR = 128
NBUF = 3
NU = S * A * (BT // GR)


def _make_sc_kernel():
    u_per_w = NU // NW
    mesh = plsc.VectorSubcoreMesh(core_axis_name="c", subcore_axis_name="s")

    @functools.partial(
        pl.kernel,
        out_type=jax.ShapeDtypeStruct((S, A, 8, BT // GR, 8, GR), jnp.float32),
        mesh=mesh,
        compiler_params=pltpu.CompilerParams(use_tc_tiling_on_sc=False,
                                             needs_layout_passes=False),
        scratch_types=[
            pltpu.VMEM((NU // NW, GR), jnp.int32),
            [pltpu.VMEM((GR, D), jnp.float32) for _ in range(NBUF)],
            [pltpu.VMEM((8, 8, GR + 1), jnp.float32) for _ in range(NBUF)],
            pltpu.VMEM((A, D), jnp.float32),
            [pltpu.SemaphoreType.DMA for _ in range(NBUF)],
            [pltpu.SemaphoreType.DMA for _ in range(NBUF)],
        ],
    )
    def body(xl_hbm, emb_hbm, pos_hbm, out_hbm,
             idx_v, g_bufs, o_bufs, pos_v, gsem, osem):
        wid = lax.axis_index("s") * NC + lax.axis_index("c")
        q0 = pl.multiple_of(wid * u_per_w, u_per_w)
        pltpu.sync_copy(pos_hbm.at[pl.ds(0, A)], pos_v)
        pltpu.sync_copy(xl_hbm.at[pl.ds(q0, u_per_w)], idx_v)

        lanes = lax.iota(jnp.int32, 16)
        iv = lanes & 7
        rvecs = [lax.shift_right_logical(lanes, 3) + dg * 2
                 for dg in range(D // 16)]

        def fire_gather(u, k):
            pltpu.async_copy(emb_hbm.at[idx_v.at[u]], g_bufs[k], gsem[k])

        def wait_gather(k):
            pltpu.make_async_copy(emb_hbm.at[pl.ds(0, GR)], g_bufs[k],
                                  gsem[k]).wait()

        def fire_out(u, k):
            q = q0 + u
            s = q // (A * (BT // GR))
            a = (q // (BT // GR)) % A
            c = q % (BT // GR)
            pltpu.async_copy(o_bufs[k].at[:, :, pl.ds(0, GR)],
                             out_hbm.at[s, a, :, c], osem[k])

        def wait_out(k):
            pltpu.make_async_copy(out_hbm.at[0, 0, :, 0],
                                  o_bufs[k].at[:, :, pl.ds(0, GR)],
                                  osem[k]).wait()

        def transpose_add(u, k):
            q = q0 + u
            a = (q // (BT // GR)) % A
            pvecs = [pos_v[a, pl.ds(dg * 16, 16)] for dg in range(D // 16)]

            def b_body(b, _, k=k, pvecs=pvecs):
                cvec = jnp.full((16,), b, jnp.int32)
                for dg in range(D // 16):
                    row = g_bufs[k][b, pl.ds(dg * 16, 16)] + pvecs[dg]
                    plsc.store_scatter(o_bufs[k], [rvecs[dg], iv, cvec], row)
                return 0

            lax.fori_loop(0, GR, b_body, 0, unroll=4)

        for k in range(NBUF - 1):
            fire_gather(k, k)

        def step_body(t, carry):
            for k in range(NBUF):
                u = t * NBUF + k
                @pl.when(u >= NBUF)
                def _(k=k):
                    wait_out(k)

                kn = (k + NBUF - 1) % NBUF

                @pl.when(u + NBUF - 1 < u_per_w)
                def _(u=u, kn=kn):
                    fire_gather(u + NBUF - 1, kn)

                wait_gather(k)
                transpose_add(u, k)
                fire_out(u, k)
            return carry

        lax.fori_loop(0, u_per_w // NBUF, step_body, 0)
        for k in range(NBUF):
            wait_out(k)

    return body


def kernel(x, action_emb, action_pos_emb):
    xl = jnp.transpose(x, (1, 2, 0)).reshape(NU, GR)
    out_t = _make_sc_kernel()(xl, action_emb, action_pos_emb)
    return jnp.transpose(out_t, (3, 5, 0, 1, 2, 4)).reshape(BT, S, A, D)
